# baseline (speedup 1.0000x reference)
import functools

import jax
import jax.numpy as jnp
from jax import lax
from jax.experimental import pallas as pl
from jax.experimental.pallas import tpu as pltpu
from jax.experimental.pallas import tpu_sc as plsc

N = 10000
D = 128
E = 320000
NC = 2
NS = 16
NW = NC * NS
N_PAD = 10240
E_PAD = 327680
EPT = E_PAD // NW
CH = 128
NCHUNK = EPT // CH
ECH = 80
ENCHUNK = EPT // ECH
RPT = N_PAD // NS
ZB = 128

BLK = 1280
GRID = N_PAD // BLK


def _sc_mesh():
    return plsc.VectorSubcoreMesh(core_axis_name="c", subcore_axis_name="s",
                                  num_cores=NC, num_subcores=NS)


def _fill_zeros(buf):
    def outer(i, _):
        def inner(j, _):
            buf[i, pl.ds(j * 16, 16)] = jnp.zeros((16,), jnp.float32)
            return 0

        lax.fori_loop(0, D // 16, inner, 0)
        return 0

    lax.fori_loop(0, ZB, outer, 0)


def _zero_acc(zeros_v, acc_sh, s):
    def body(j, _):
        pltpu.sync_copy(zeros_v, acc_sh.at[pl.ds(s * RPT + j * ZB, ZB)])
        return 0

    lax.fori_loop(0, RPT // ZB, body, 0)


def _write_out(acc_sh, out_hbm, c, s):
    def body(j, _):
        r0 = s * RPT + j * ZB
        pltpu.sync_copy(acc_sh.at[pl.ds(r0, ZB)], out_hbm.at[c, pl.ds(r0, ZB)])
        return 0

    lax.fori_loop(0, RPT // ZB, body, 0)


def _deg_body(col_hbm, out_hbm, idx_v, ones_v, zeros_v, acc_sh):
    c = lax.axis_index("c")
    s = lax.axis_index("s")
    wid = s * NC + c

    _fill_zeros(zeros_v)

    def fill_ones(i, _):
        def inner(j, _):
            ones_v[i, pl.ds(j * 16, 16)] = jnp.full((16,), 1.0, jnp.float32)
            return 0

        lax.fori_loop(0, D // 16, inner, 0)
        return 0

    lax.fori_loop(0, CH, fill_ones, 0)
    _zero_acc(zeros_v, acc_sh, s)
    plsc.subcore_barrier()

    base = wid * EPT

    def body(i, _):
        pltpu.sync_copy(col_hbm.at[pl.ds(base + i * CH, CH)], idx_v)
        pltpu.sync_copy(ones_v, acc_sh.at[idx_v], add=True)
        return 0

    lax.fori_loop(0, NCHUNK, body, 0)

    plsc.subcore_barrier()
    _write_out(acc_sh, out_hbm, c, s)


@functools.cache
def _make_deg_kernel():
    return functools.partial(
        pl.kernel,
        out_type=jax.ShapeDtypeStruct((NC, N_PAD, D), jnp.float32),
        mesh=_sc_mesh(),
        scratch_types=[
            pltpu.VMEM((CH,), jnp.int32),
            pltpu.VMEM((CH, D), jnp.float32),
            pltpu.VMEM((ZB, D), jnp.float32),
            pltpu.VMEM_SHARED((N_PAD, D), jnp.float32),
        ],
    )(_deg_body)


NBUF = 4


def _edge_body(h_hbm, row_hbm, col_hbm, out_hbm,
               colv, rowv, rows_v, acc_sh, gsem, isem, ssem):
    c = lax.axis_index("c")
    s = lax.axis_index("s")
    wid = s * NC + c

    def fz(i, _):
        def inner(j, _):
            rows_v[0][i, pl.ds(j * 16, 16)] = jnp.zeros((16,), jnp.float32)
            return 0
        lax.fori_loop(0, D // 16, inner, 0)
        return 0
    lax.fori_loop(0, ECH, fz, 0)

    def zc(j, _):
        pltpu.sync_copy(rows_v[0], acc_sh.at[pl.ds(s * RPT + j * ECH, ECH)])
        return 0
    lax.fori_loop(0, RPT // ECH, zc, 0)
    plsc.subcore_barrier()

    base = wid * EPT

    def idx_start(i, b):
        off = base + i * ECH
        pltpu.async_copy(col_hbm.at[pl.ds(off, ECH)], colv[b], isem[b])
        pltpu.async_copy(row_hbm.at[pl.ds(off, ECH)], rowv[b], isem[b])

    def idx_wait(b):
        pltpu.make_async_copy(col_hbm.at[pl.ds(0, ECH)], colv[b],
                              isem[b]).wait()
        pltpu.make_async_copy(row_hbm.at[pl.ds(0, ECH)], rowv[b],
                              isem[b]).wait()

    def gather_start(b):
        pltpu.async_copy(h_hbm.at[colv[b]], rows_v[b], gsem[b])

    def gather_wait(b):
        pltpu.make_async_copy(h_hbm.at[colv[b]], rows_v[b],
                              gsem[b]).wait()

    def scatter_start(b):
        pltpu.async_copy(rows_v[b], acc_sh.at[rowv[b]], ssem[b],
                         add=True)

    def scatter_wait(b):
        pltpu.make_async_copy(rows_v[b], acc_sh.at[rowv[b]],
                              ssem[b]).wait()

    idx_start(0, 0)
    idx_wait(0)
    gather_start(0)

    def quad(k2, _):
        for j in range(NBUF):
            k = NBUF * k2 + j
            nxt = (j + 1) % NBUF

            if j < NBUF - 1:
                @pl.when(k2 > 0)
                def _():
                    scatter_wait(nxt)
            else:
                scatter_wait(nxt)

            if j < NBUF - 1:
                idx_start(k + 1, nxt)
                gather_wait(j)
                scatter_start(j)
                idx_wait(nxt)
                gather_start(nxt)
            else:
                @pl.when(k2 < ENCHUNK // NBUF - 1)
                def _():
                    idx_start(k + 1, nxt)
                gather_wait(j)
                scatter_start(j)

                @pl.when(k2 < ENCHUNK // NBUF - 1)
                def _():
                    idx_wait(nxt)
                    gather_start(nxt)
        return 0

    lax.fori_loop(0, ENCHUNK // NBUF, quad, 0)

    for b in range(1, NBUF):
        scatter_wait(b)

    plsc.subcore_barrier()

    def wo(j, _):
        r0 = s * RPT + j * ECH
        pltpu.sync_copy(acc_sh.at[pl.ds(r0, ECH)],
                        out_hbm.at[c, pl.ds(r0, ECH)])
        return 0

    lax.fori_loop(0, RPT // ECH, wo, 0)


@functools.cache
def _make_edge_kernel():
    return functools.partial(
        pl.kernel,
        out_type=jax.ShapeDtypeStruct((NC, N_PAD, D), jnp.float32),
        mesh=_sc_mesh(),
        scratch_types=[
            [pltpu.VMEM((ECH,), jnp.int32)] * NBUF,
            [pltpu.VMEM((ECH,), jnp.int32)] * NBUF,
            [pltpu.VMEM((ECH, D), jnp.float32)] * NBUF,
            pltpu.VMEM_SHARED((N_PAD, D), jnp.float32),
            [pltpu.SemaphoreType.DMA] * NBUF,
            [pltpu.SemaphoreType.DMA] * NBUF,
            [pltpu.SemaphoreType.DMA] * NBUF,
        ],
    )(_edge_body)


def _build_dense(n_parts):
    def body(p_ref, wt_ref, b_ref, degp_ref, out_ref):
        h = p_ref[0]
        for q in range(1, n_parts):
            h = h + p_ref[q]
        g = jnp.dot(h, wt_ref[...], preferred_element_type=jnp.float32)
        g = jnp.maximum(g + b_ref[...], 0.0)
        deg = degp_ref[0] + degp_ref[1]
        deg = jnp.maximum(deg, 1.0)
        out_ref[...] = g / deg

    return pl.pallas_call(
        body,
        grid=(GRID,),
        in_specs=[
            pl.BlockSpec((n_parts, BLK, D), lambda i: (0, i, 0)),
            pl.BlockSpec((D, D), lambda i: (0, 0)),
            pl.BlockSpec((1, D), lambda i: (0, 0)),
            pl.BlockSpec((NC, BLK, D), lambda i: (0, i, 0)),
        ],
        out_specs=pl.BlockSpec((BLK, D), lambda i: (i, 0)),
        out_shape=jax.ShapeDtypeStruct((N_PAD, D), jnp.float32),
    )


_dense_1 = _build_dense(1)
_dense_2 = _build_dense(NC)


def _head_body(p_ref, pw0, pb0, fw0, fb0, pw1, pb1, fw1, fb1, ow, ob, out_ref):
    h3 = p_ref[0] + p_ref[1]
    inj0 = jnp.dot(h3, pw0[...], preferred_element_type=jnp.float32) + pb0[...]
    h = jnp.dot(jnp.maximum(h3, 0.0), fw0[...],
                preferred_element_type=jnp.float32) + fb0[...] + inj0
    inj1 = jnp.dot(h3, pw1[...], preferred_element_type=jnp.float32) + pb1[...]
    h = jnp.dot(jnp.maximum(h, 0.0), fw1[...],
                preferred_element_type=jnp.float32) + fb1[...] + inj1
    out_ref[...] = jnp.dot(h, ow[...], preferred_element_type=jnp.float32) + ob[...]


_mat_spec = pl.BlockSpec((D, D), lambda i: (0, 0))
_vec_spec = pl.BlockSpec((1, D), lambda i: (0, 0))

_head = pl.pallas_call(
    _head_body,
    grid=(GRID,),
    in_specs=[pl.BlockSpec((NC, BLK, D), lambda i: (0, i, 0)),
              _mat_spec, _vec_spec, _mat_spec, _vec_spec,
              _mat_spec, _vec_spec, _mat_spec, _vec_spec,
              _mat_spec, _vec_spec],
    out_specs=pl.BlockSpec((BLK, D), lambda i: (i, 0)),
    out_shape=jax.ShapeDtypeStruct((N_PAD, D), jnp.float32),
)


def kernel(x, edge_index, mp_W0, mp_b0, mp_W1, mp_b1, mp_W2, mp_b2,
           fc_W0, fc_b0, fc_W1, fc_b1, proj_W0, proj_b0, proj_W1, proj_b1,
           out_W, out_b, alpha):
    row = edge_index[0]
    col = edge_index[1]

    x_pad = jnp.concatenate(
        [x, jnp.zeros((N_PAD - N, D), jnp.float32)], axis=0)
    epad = jnp.full((E_PAD - E,), N, jnp.int32)
    row_p = jnp.concatenate([row, epad])
    col_p = jnp.concatenate([col, epad])

    deg_kernel = _make_deg_kernel()
    edge_kernel = _make_edge_kernel()

    degp = deg_kernel(col_p)

    g = _dense_1(x_pad[None], mp_W0.T, mp_b0[None], degp)
    p = edge_kernel(g, row_p, col_p)
    g = _dense_2(p, mp_W1.T, mp_b1[None], degp)
    p = edge_kernel(g, row_p, col_p)
    g = _dense_2(p, mp_W2.T, mp_b2[None], degp)
    p = edge_kernel(g, row_p, col_p)

    out = _head(p,
                proj_W0.T, proj_b0[None],
                (alpha * fc_W0).T, (alpha * fc_b0)[None],
                proj_W1.T, proj_b1[None],
                (alpha * fc_W1).T, (alpha * fc_b1)[None],
                out_W.T, out_b[None])
    return out[:N]


# speedup vs baseline: 2.8331x; 2.8331x over previous
import functools

import jax
import jax.numpy as jnp
from jax import lax
from jax.experimental import pallas as pl
from jax.experimental.pallas import tpu as pltpu
from jax.experimental.pallas import tpu_sc as plsc

N = 10000
D = 128
E = 320000
NC = 2
NS = 16
NW = NC * NS
N_PAD = 10240
E_PAD = 327680
EPT = E_PAD // NW
CH = 128
NCHUNK = EPT // CH
ECH = 128
ENCHUNK = EPT // ECH
RPT = N_PAD // NS
ZB = 128

BLK = 1280
GRID = N_PAD // BLK


def _sc_mesh():
    return plsc.VectorSubcoreMesh(core_axis_name="c", subcore_axis_name="s",
                                  num_cores=NC, num_subcores=NS)


def _fill_zeros(buf):
    def outer(i, _):
        def inner(j, _):
            buf[i, pl.ds(j * 16, 16)] = jnp.zeros((16,), jnp.float32)
            return 0

        lax.fori_loop(0, D // 16, inner, 0)
        return 0

    lax.fori_loop(0, ZB, outer, 0)


def _zero_acc(zeros_v, acc_sh, s):
    def body(j, _):
        pltpu.sync_copy(zeros_v, acc_sh.at[pl.ds(s * RPT + j * ZB, ZB)])
        return 0

    lax.fori_loop(0, RPT // ZB, body, 0)


def _write_out(acc_sh, out_hbm, c, s):
    def body(j, _):
        r0 = s * RPT + j * ZB
        pltpu.sync_copy(acc_sh.at[pl.ds(r0, ZB)], out_hbm.at[c, pl.ds(r0, ZB)])
        return 0

    lax.fori_loop(0, RPT // ZB, body, 0)


def _deg_body(col_hbm, out_hbm, idx_v, ones_v, zeros_v, acc_sh):
    c = lax.axis_index("c")
    s = lax.axis_index("s")
    wid = s * NC + c

    _fill_zeros(zeros_v)

    def fill_ones(i, _):
        def inner(j, _):
            ones_v[i, pl.ds(j * 16, 16)] = jnp.full((16,), 1.0, jnp.float32)
            return 0

        lax.fori_loop(0, D // 16, inner, 0)
        return 0

    lax.fori_loop(0, CH, fill_ones, 0)
    _zero_acc(zeros_v, acc_sh, s)
    plsc.subcore_barrier()

    base = wid * EPT

    def body(i, _):
        pltpu.sync_copy(col_hbm.at[pl.ds(base + i * CH, CH)], idx_v)
        pltpu.sync_copy(ones_v, acc_sh.at[idx_v], add=True)
        return 0

    lax.fori_loop(0, NCHUNK, body, 0)

    plsc.subcore_barrier()
    _write_out(acc_sh, out_hbm, c, s)


@functools.cache
def _make_deg_kernel():
    return functools.partial(
        pl.kernel,
        out_type=jax.ShapeDtypeStruct((NC, N_PAD, D), jnp.float32),
        mesh=_sc_mesh(),
        scratch_types=[
            pltpu.VMEM((CH,), jnp.int32),
            pltpu.VMEM((CH, D), jnp.float32),
            pltpu.VMEM((ZB, D), jnp.float32),
            pltpu.VMEM_SHARED((N_PAD, D), jnp.float32),
        ],
    )(_deg_body)


NBUF = 2


def _edge_body(h_hbm, row_hbm, col_hbm, out_hbm,
               colv, rowv, rows_v, acc_sh, gsem, isem, ssem):
    c = lax.axis_index("c")
    s = lax.axis_index("s")
    wid = s * NC + c

    def fz(i, _):
        def inner(j, _):
            rows_v[0][i, pl.ds(j * 16, 16)] = jnp.zeros((16,), jnp.float32)
            return 0
        lax.fori_loop(0, D // 16, inner, 0)
        return 0
    lax.fori_loop(0, ECH, fz, 0)

    def zc(j, _):
        pltpu.sync_copy(rows_v[0], acc_sh.at[pl.ds(s * RPT + j * ECH, ECH)])
        return 0
    lax.fori_loop(0, RPT // ECH, zc, 0)
    plsc.subcore_barrier()

    base = wid * EPT

    def idx_start(i, b):
        off = base + i * ECH
        pltpu.async_copy(col_hbm.at[pl.ds(off, ECH)], colv[b], isem[b])
        pltpu.async_copy(row_hbm.at[pl.ds(off, ECH)], rowv[b], isem[b])

    def idx_wait(b):
        pltpu.make_async_copy(col_hbm.at[pl.ds(0, ECH)], colv[b],
                              isem[b]).wait()
        pltpu.make_async_copy(row_hbm.at[pl.ds(0, ECH)], rowv[b],
                              isem[b]).wait()

    def gather_start(b):
        pltpu.async_copy(h_hbm.at[colv[b]], rows_v[b], gsem[b])

    def gather_wait(b):
        pltpu.make_async_copy(h_hbm.at[colv[b]], rows_v[b],
                              gsem[b]).wait()

    def scatter_start(b):
        pltpu.async_copy(rows_v[b], acc_sh.at[rowv[b]], ssem[b],
                         add=True)

    def scatter_wait(b):
        pltpu.make_async_copy(rows_v[b], acc_sh.at[rowv[b]],
                              ssem[b]).wait()

    idx_start(0, 0)
    idx_wait(0)
    gather_start(0)

    def quad(k2, _):
        for j in range(NBUF):
            k = NBUF * k2 + j
            nxt = (j + 1) % NBUF

            if j < NBUF - 1:
                @pl.when(k2 > 0)
                def _():
                    scatter_wait(nxt)
            else:
                scatter_wait(nxt)

            if j < NBUF - 1:
                idx_start(k + 1, nxt)
                gather_wait(j)
                scatter_start(j)
                idx_wait(nxt)
                gather_start(nxt)
            else:
                @pl.when(k2 < ENCHUNK // NBUF - 1)
                def _():
                    idx_start(k + 1, nxt)
                gather_wait(j)
                scatter_start(j)

                @pl.when(k2 < ENCHUNK // NBUF - 1)
                def _():
                    idx_wait(nxt)
                    gather_start(nxt)
        return 0

    lax.fori_loop(0, ENCHUNK // NBUF, quad, 0)

    for b in range(1, NBUF):
        scatter_wait(b)

    plsc.subcore_barrier()

    def wo(j, _):
        r0 = s * RPT + j * ECH
        pltpu.sync_copy(acc_sh.at[pl.ds(r0, ECH)],
                        out_hbm.at[c, pl.ds(r0, ECH)])
        return 0

    lax.fori_loop(0, RPT // ECH, wo, 0)


@functools.cache
def _make_edge_kernel():
    return functools.partial(
        pl.kernel,
        out_type=jax.ShapeDtypeStruct((NC, N_PAD, D), jnp.float32),
        mesh=_sc_mesh(),
        scratch_types=[
            [pltpu.VMEM((ECH,), jnp.int32)] * NBUF,
            [pltpu.VMEM((ECH,), jnp.int32)] * NBUF,
            [pltpu.VMEM((ECH, D), jnp.float32)] * NBUF,
            pltpu.VMEM_SHARED((N_PAD, D), jnp.float32),
            [pltpu.SemaphoreType.DMA] * NBUF,
            [pltpu.SemaphoreType.DMA] * NBUF,
            [pltpu.SemaphoreType.DMA] * NBUF,
        ],
    )(_edge_body)


def _build_dense(n_parts):
    def body(p_ref, wt_ref, b_ref, degp_ref, out_ref):
        h = p_ref[0]
        for q in range(1, n_parts):
            h = h + p_ref[q]
        g = jnp.dot(h, wt_ref[...], preferred_element_type=jnp.float32)
        g = jnp.maximum(g + b_ref[...], 0.0)
        deg = degp_ref[0] + degp_ref[1]
        deg = jnp.maximum(deg, 1.0)
        out_ref[...] = g / deg

    return pl.pallas_call(
        body,
        grid=(GRID,),
        in_specs=[
            pl.BlockSpec((n_parts, BLK, D), lambda i: (0, i, 0)),
            pl.BlockSpec((D, D), lambda i: (0, 0)),
            pl.BlockSpec((1, D), lambda i: (0, 0)),
            pl.BlockSpec((NC, BLK, D), lambda i: (0, i, 0)),
        ],
        out_specs=pl.BlockSpec((BLK, D), lambda i: (i, 0)),
        out_shape=jax.ShapeDtypeStruct((N_PAD, D), jnp.float32),
    )


_dense_1 = _build_dense(1)
_dense_2 = _build_dense(NC)


def _head_body(p_ref, pw0, pb0, fw0, fb0, pw1, pb1, fw1, fb1, ow, ob, out_ref):
    h3 = p_ref[0] + p_ref[1]
    inj0 = jnp.dot(h3, pw0[...], preferred_element_type=jnp.float32) + pb0[...]
    h = jnp.dot(jnp.maximum(h3, 0.0), fw0[...],
                preferred_element_type=jnp.float32) + fb0[...] + inj0
    inj1 = jnp.dot(h3, pw1[...], preferred_element_type=jnp.float32) + pb1[...]
    h = jnp.dot(jnp.maximum(h, 0.0), fw1[...],
                preferred_element_type=jnp.float32) + fb1[...] + inj1
    out_ref[...] = jnp.dot(h, ow[...], preferred_element_type=jnp.float32) + ob[...]


_mat_spec = pl.BlockSpec((D, D), lambda i: (0, 0))
_vec_spec = pl.BlockSpec((1, D), lambda i: (0, 0))

_head = pl.pallas_call(
    _head_body,
    grid=(GRID,),
    in_specs=[pl.BlockSpec((NC, BLK, D), lambda i: (0, i, 0)),
              _mat_spec, _vec_spec, _mat_spec, _vec_spec,
              _mat_spec, _vec_spec, _mat_spec, _vec_spec,
              _mat_spec, _vec_spec],
    out_specs=pl.BlockSpec((BLK, D), lambda i: (i, 0)),
    out_shape=jax.ShapeDtypeStruct((N_PAD, D), jnp.float32),
)


def kernel(x, edge_index, mp_W0, mp_b0, mp_W1, mp_b1, mp_W2, mp_b2,
           fc_W0, fc_b0, fc_W1, fc_b1, proj_W0, proj_b0, proj_W1, proj_b1,
           out_W, out_b, alpha):
    row = edge_index[0]
    col = edge_index[1]

    x_pad = jnp.concatenate(
        [x, jnp.zeros((N_PAD - N, D), jnp.float32)], axis=0)
    ppt = EPT - E // NW
    pad_blk = jnp.broadcast_to(
        N + jnp.arange(ppt, dtype=jnp.int32), (NW, ppt))
    row_p = jnp.concatenate(
        [row.reshape(NW, E // NW), pad_blk], axis=1).reshape(-1)
    col_p = jnp.concatenate(
        [col.reshape(NW, E // NW), pad_blk], axis=1).reshape(-1)

    deg_kernel = _make_deg_kernel()
    edge_kernel = _make_edge_kernel()

    degp = deg_kernel(col_p)

    g = _dense_1(x_pad[None], mp_W0.T, mp_b0[None], degp)
    p = edge_kernel(g, row_p, col_p)
    g = _dense_2(p, mp_W1.T, mp_b1[None], degp)
    p = edge_kernel(g, row_p, col_p)
    g = _dense_2(p, mp_W2.T, mp_b2[None], degp)
    p = edge_kernel(g, row_p, col_p)

    out = _head(p,
                proj_W0.T, proj_b0[None],
                (alpha * fc_W0).T, (alpha * fc_b0)[None],
                proj_W1.T, proj_b1[None],
                (alpha * fc_W1).T, (alpha * fc_b1)[None],
                out_W.T, out_b[None])
    return out[:N]
